# baseline (speedup 1.0000x reference)
import dataclasses
import functools

import jax
import jax.numpy as jnp
from jax import lax
from jax.experimental import pallas as pl
from jax.experimental.pallas import tpu as pltpu
from jax.experimental.pallas import tpu_sc as plsc

_N = 100000
_D = 64


def kernel(x, table):
    idx = x.reshape(_N)
    tab_flat = jnp.pad(table.T, ((0, 0), (0, 11))).reshape(32 * _D)
    mesh = plsc.VectorSubcoreMesh(core_axis_name="c", subcore_axis_name="s")
    cp = pltpu.CompilerParams()
    if "needs_layout_passes" in pltpu.CompilerParams.__dataclass_fields__:
        cp = dataclasses.replace(cp, needs_layout_passes=False)

    @functools.partial(
        pl.kernel,
        out_type=jax.ShapeDtypeStruct((_D, _N), table.dtype),
        mesh=mesh,
        compiler_params=cp,
        scratch_types=[
            pltpu.VMEM((_D, 128), jnp.float32),
        ],
    )
    def gather_kernel(tab_hbm, idx_hbm, out_hbm, blk_v):
        wid = lax.axis_index("s") * 2 + lax.axis_index("c")
        _NW, _NFULL, _C, _K = 32, 781, 128, 26

        @pl.loop(0, _K)
        def _(k):
            c_raw = wid + k * _NW
            base = jnp.where(c_raw < _NFULL, c_raw, wid) * _C
            pltpu.sync_copy(blk_v, out_hbm.at[:, pl.ds(base, _C)])

    return gather_kernel(tab_flat, idx).T
